# Initial kernel scaffold; baseline (speedup 1.0000x reference)
import functools
import numpy as np
import jax
import jax.numpy as jnp
from jax import lax
from jax.experimental import pallas as pl
from jax.experimental.pallas import tpu as pltpu
from jax.experimental.pallas import tpu_sc as plsc

N = 10000
E = 320000
B = 8
D = 128
NCLS = 16
SEG = N // B
HSIZE = N * NCLS
HPAD = HSIZE + 8

NUM_CORES = 2
NUM_SUBCORES = 16
W = NUM_CORES * NUM_SUBCORES
EPW = E // W
IDX_COLS = 128
FULL_ROWS = EPW // IDX_COLS
IDX_ROWS = FULL_ROWS + 1

COLS = 1280
TRI = SEG * (SEG - 1) // 2
OUTP = 781952
ROWT = 50


def _hist_body(xcls_hbm, edges_hbm, out_hbm,
               xcls_v, src_v, dst_v, idx_v, upd_v, zero_v, hist_sh, sem):
    c = lax.axis_index("c")
    s = lax.axis_index("s")
    wid = c * NUM_SUBCORES + s
    base = wid * EPW
    sbase = s * (HSIZE // NUM_SUBCORES)

    pltpu.sync_copy(xcls_hbm, xcls_v)
    pltpu.sync_copy(edges_hbm.at[0, pl.ds(base, EPW)], src_v)
    pltpu.sync_copy(edges_hbm.at[1, pl.ds(base, EPW)], dst_v)

    ones16 = jnp.full((16,), 1.0, jnp.float32)
    zeros16 = jnp.zeros((16,), jnp.float32)
    trash16 = jnp.full((16,), HSIZE, jnp.int32)

    def zfill(i, _):
        zero_v[pl.ds(i * 16, 16)] = zeros16
        return 0
    lax.fori_loop(0, (HSIZE // NUM_SUBCORES) // 16, zfill, 0)
    pltpu.sync_copy(zero_v, hist_sh.at[pl.ds(sbase, HSIZE // NUM_SUBCORES)])

    def ifill(i, _):
        r = i // 8
        col = (i % 8) * 16
        upd_v[r, pl.ds(col, 16)] = ones16
        idx_v[r, pl.ds(col, 16)] = trash16
        return 0
    lax.fori_loop(0, IDX_ROWS * 8, ifill, 0)

    def efill(j, _):
        sv = src_v[pl.ds(j * 16, 16)]
        dv = dst_v[pl.ds(j * 16, 16)]
        cls = plsc.load_gather(xcls_v, [sv])
        flat = dv * 16 + cls
        r = j // 8
        col = (j % 8) * 16
        idx_v[r, pl.ds(col, 16)] = flat
        return 0
    lax.fori_loop(0, EPW // 16, efill, 0)

    plsc.subcore_barrier()

    def scat(r, _):
        pltpu.sync_copy(upd_v.at[r], hist_sh.at[idx_v.at[r]], add=True)
        return 0
    lax.fori_loop(0, IDX_ROWS, scat, 0)

    plsc.subcore_barrier()

    pltpu.sync_copy(hist_sh.at[pl.ds(sbase, HSIZE // NUM_SUBCORES)],
                    out_hbm.at[c, pl.ds(sbase, HSIZE // NUM_SUBCORES)])


def _histogram(xcls, edges):
    mesh = plsc.VectorSubcoreMesh(core_axis_name="c", subcore_axis_name="s")
    f = functools.partial(
        pl.kernel,
        mesh=mesh,
        out_type=jax.ShapeDtypeStruct((NUM_CORES, HSIZE), jnp.float32),
        scratch_types=[
            pltpu.VMEM((N,), jnp.int32),
            pltpu.VMEM((EPW,), jnp.int32),
            pltpu.VMEM((EPW,), jnp.int32),
            pltpu.VMEM((IDX_ROWS, IDX_COLS), jnp.int32),
            pltpu.VMEM((IDX_ROWS, IDX_COLS), jnp.float32),
            pltpu.VMEM((HSIZE // NUM_SUBCORES,), jnp.float32),
            pltpu.VMEM_SHARED((HPAD,), jnp.float32),
            pltpu.SemaphoreType.DMA,
        ],
    )(_hist_body)
    return f(xcls, edges)


def _ln(h, g, b):
    mu = jnp.mean(h, axis=-1, keepdims=True)
    var = jnp.mean((h - mu) * (h - mu), axis=-1, keepdims=True)
    return (h - mu) * jax.lax.rsqrt(var + 1e-5) * g + b


def _main_body(part_ref, xc_ref, emb_ref,
               w1, b1, g1, be1, w2, b2, g2, be2, w3, b3,
               sw1, sb1, sw2, sb2, ng, nb,
               xf_ref, gm_ref):
    cnt = part_ref[0, 0] + part_ref[1, 0]
    cls = xc_ref[0]
    onehot = (cls == lax.broadcasted_iota(jnp.int32, (SEG, NCLS), 1))
    c = cnt + onehot.astype(jnp.float32)
    h = jnp.dot(c, emb_ref[...], preferred_element_type=jnp.float32)
    t = jnp.dot(h, w1[...], preferred_element_type=jnp.float32) + b1[...]
    t = jnp.maximum(_ln(t, g1[...], be1[...]), 0.0)
    t = jnp.dot(t, w2[...], preferred_element_type=jnp.float32) + b2[...]
    t = jnp.maximum(_ln(t, g2[...], be2[...]), 0.0)
    t = jnp.dot(t, w3[...], preferred_element_type=jnp.float32) + b3[...]
    t = jnp.maximum(jnp.dot(t, sw1[...], preferred_element_type=jnp.float32)
                    + sb1[...], 0.0)
    t = jnp.dot(t, sw2[...], preferred_element_type=jnp.float32) + sb2[...]
    xf = _ln(t, ng[...], nb[...])
    xf_ref[0] = xf
    gm_ref[0] = jnp.mean(xf, axis=0)


def _main(parts, xcls, p):
    g = p["gin_mlp"]
    row = lambda v: v.reshape(1, -1)
    args = [
        parts.reshape(NUM_CORES, B, SEG, NCLS),
        xcls.reshape(B, SEG, 1),
        p["embedding"],
        g["W1"], row(g["b1"]), row(g["g1"]), row(g["be1"]),
        g["W2"], row(g["b2"]), row(g["g2"]), row(g["be2"]),
        g["W3"], row(g["b3"]),
        p["seq_W1"], row(p["seq_b1"]), p["seq_W2"], row(p["seq_b2"]),
        row(p["norm_g"]), row(p["norm_b"]),
    ]

    def full(a):
        nd = a.ndim
        return pl.BlockSpec(a.shape, lambda i, nd=nd: (0,) * nd)

    specs = [
        pl.BlockSpec((NUM_CORES, 1, SEG, NCLS), lambda i: (0, i, 0, 0)),
        pl.BlockSpec((1, SEG, 1), lambda i: (i, 0, 0)),
    ] + [full(a) for a in args[2:]]
    return pl.pallas_call(
        _main_body,
        grid=(B,),
        in_specs=specs,
        out_specs=[
            pl.BlockSpec((1, SEG, D), lambda i: (i, 0, 0)),
            pl.BlockSpec((1, D), lambda i: (i, 0)),
        ],
        out_shape=[
            jax.ShapeDtypeStruct((B, SEG, D), jnp.float32),
            jax.ShapeDtypeStruct((B, D), jnp.float32),
        ],
    )(*args)


def _heads_body(gm, w1s, b1s, g1s, be1s, w2s, b2s, g2s, be2s, w3s, b3s,
                act_ref, nc_ref, ec_ref):
    x = gm[...]
    outs = []
    for i in range(4):
        t = jnp.dot(x, w1s[i], preferred_element_type=jnp.float32) + b1s[i]
        t = jnp.maximum(_ln(t, g1s[i], be1s[i]), 0.0)
        t = jnp.dot(t, w2s[i], preferred_element_type=jnp.float32) + b2s[i]
        t = jnp.maximum(_ln(t, g2s[i], be2s[i]), 0.0)
        outs.append(jnp.dot(t, w3s[i], preferred_element_type=jnp.float32)
                    + b3s[i])
    addn, exitl, ncl, ecl = outs
    zero = jnp.zeros((B, 1), jnp.float32)
    act_ref[...] = jnp.concatenate(
        [addn[:, 0:1], zero, exitl[:, 0:1]], axis=1)
    nc_ref[...] = ncl
    ec_ref[...] = ecl[:, 0:4]


def _heads(gm, p):
    mlps = [p["add_node_mlp"], p["exit_mlp"], p["node_class_mlp"],
            p["edge_class_mlp"]]
    st = lambda k: jnp.stack([m[k] for m in mlps])
    w3s = jnp.stack([
        jnp.pad(m["W3"], ((0, 0), (0, NCLS - m["W3"].shape[1])))
        for m in mlps])
    b3s = jnp.stack([jnp.pad(m["b3"], (0, NCLS - m["b3"].shape[0]))
                     for m in mlps])
    args = [gm, st("W1"), st("b1"), st("g1"), st("be1"),
            st("W2"), st("b2"), st("g2"), st("be2"), w3s, b3s]
    return pl.pallas_call(
        _heads_body,
        out_shape=[
            jax.ShapeDtypeStruct((B, 3), jnp.float32),
            jax.ShapeDtypeStruct((B, NCLS), jnp.float32),
            jax.ShapeDtypeStruct((B, 4), jnp.float32),
        ],
    )(*args)


def _scores_body(xf_ref, out_ref):
    xg = xf_ref[0]
    s = lax.dot_general(xg, xg, (((1,), (1,)), ((), ())),
                        preferred_element_type=jnp.float32)
    out_ref[0, :, pl.ds(0, SEG)] = s * np.float32(1.0 / np.sqrt(D))


def _scores(xf):
    return pl.pallas_call(
        _scores_body,
        grid=(B,),
        in_specs=[pl.BlockSpec((1, SEG, D), lambda i: (i, 0, 0))],
        out_specs=pl.BlockSpec((1, SEG, COLS), lambda i: (i, 0, 0)),
        out_shape=jax.ShapeDtypeStruct((B, SEG, COLS), jnp.float32),
    )(xf)


def _extract_body(s_ref, out_ref):
    r0 = pl.program_id(0) * ROWT

    def body(t, _):
        i = r0 + t
        rows = s_ref[:, t, :]
        shifted = pltpu.roll(rows, COLS - 1 - i, axis=1)
        off = i * (SEG - 1) - (i * (i - 1)) // 2
        cur = out_ref[:, pl.ds(off, COLS)]
        keep = lax.broadcasted_iota(jnp.int32, (B, COLS), 1) < (SEG - 1 - i)
        out_ref[:, pl.ds(off, COLS)] = jnp.where(keep, shifted, cur)
        return 0

    lax.fori_loop(0, ROWT, body, 0)


def _extract(spad):
    return pl.pallas_call(
        _extract_body,
        grid=(SEG // ROWT,),
        in_specs=[pl.BlockSpec((B, ROWT, COLS), lambda i: (0, i, 0))],
        out_specs=pl.BlockSpec((B, OUTP), lambda i: (0, 0)),
        out_shape=jax.ShapeDtypeStruct((B, OUTP), jnp.float32),
        compiler_params=pltpu.CompilerParams(
            dimension_semantics=("arbitrary",)),
    )(spad)


def kernel(x, edge_index, ptr, params):
    xcls = x.reshape(N)
    parts = _histogram(xcls, edge_index)
    xf, gm = _main(parts, xcls, params)
    act, ncl, ecl = _heads(gm, params)
    spad = _scores(xf)
    logits = _extract(spad)[:, :TRI]
    return act, ncl, ecl, logits


# speedup vs baseline: 6.3719x; 6.3719x over previous
import functools
import numpy as np
import jax
import jax.numpy as jnp
from jax import lax
from jax.experimental import pallas as pl
from jax.experimental.pallas import tpu as pltpu
from jax.experimental.pallas import tpu_sc as plsc

N = 10000
E = 320000
B = 8
D = 128
NCLS = 16
SEG = N // B
HSIZE = N * NCLS
HPAD = HSIZE + 8

NUM_CORES = 2
NUM_SUBCORES = 16
W = NUM_CORES * NUM_SUBCORES
EPW = E // W
IDX_COLS = 128
FULL_ROWS = EPW // IDX_COLS
IDX_ROWS = FULL_ROWS + 1

COLS = 1280
TRI = SEG * (SEG - 1) // 2
OUTP = 781952
ROWT = 50


def _hist_body(xcls_hbm, src_hbm, dst_hbm, out_hbm,
               xcls_v, src_v, dst_v, idx_v, upd_v, zero_v, hist_sh, sem):
    c = lax.axis_index("c")
    s = lax.axis_index("s")
    wid = c * NUM_SUBCORES + s
    base = wid * EPW
    sbase = s * (HSIZE // NUM_SUBCORES)

    pltpu.sync_copy(xcls_hbm, xcls_v)
    pltpu.sync_copy(src_hbm.at[pl.ds(base, EPW)], src_v)
    pltpu.sync_copy(dst_hbm.at[pl.ds(base, EPW)], dst_v)

    ones16 = jnp.full((16,), 1.0, jnp.float32)
    zeros16 = jnp.zeros((16,), jnp.float32)
    trash16 = jnp.full((16,), HSIZE, jnp.int32)

    def zfill(i, _):
        zero_v[pl.ds(i * 16, 16)] = zeros16
        return 0
    lax.fori_loop(0, (HSIZE // NUM_SUBCORES) // 16, zfill, 0)
    pltpu.sync_copy(zero_v, hist_sh.at[pl.ds(sbase, HSIZE // NUM_SUBCORES)])

    def ifill(i, _):
        r = i // 8
        col = (i % 8) * 16
        upd_v[r, pl.ds(col, 16)] = ones16
        idx_v[r, pl.ds(col, 16)] = trash16
        return 0
    lax.fori_loop(0, IDX_ROWS * 8, ifill, 0)

    def efill(j, _):
        sv = src_v[pl.ds(j * 16, 16)]
        dv = dst_v[pl.ds(j * 16, 16)]
        cls = plsc.load_gather(xcls_v, [sv])
        flat = dv * 16 + cls
        r = j // 8
        col = (j % 8) * 16
        idx_v[r, pl.ds(col, 16)] = flat
        return 0
    lax.fori_loop(0, EPW // 16, efill, 0)

    plsc.subcore_barrier()

    def scat(r, _):
        pltpu.sync_copy(upd_v.at[r], hist_sh.at[idx_v.at[r]], add=True)
        return 0
    lax.fori_loop(0, IDX_ROWS, scat, 0)

    plsc.subcore_barrier()

    pltpu.sync_copy(hist_sh.at[pl.ds(sbase, HSIZE // NUM_SUBCORES)], zero_v)
    pltpu.sync_copy(zero_v,
                    out_hbm.at[pl.ds(c * HSIZE + sbase, HSIZE // NUM_SUBCORES)])


def _histogram(xcls, src, dst):
    mesh = plsc.VectorSubcoreMesh(core_axis_name="c", subcore_axis_name="s")
    f = functools.partial(
        pl.kernel,
        mesh=mesh,
        compiler_params=pltpu.CompilerParams(needs_layout_passes=False),
        out_type=jax.ShapeDtypeStruct((NUM_CORES * HSIZE,), jnp.float32),
        scratch_types=[
            pltpu.VMEM((N,), jnp.int32),
            pltpu.VMEM((EPW,), jnp.int32),
            pltpu.VMEM((EPW,), jnp.int32),
            pltpu.VMEM((IDX_ROWS, IDX_COLS), jnp.int32),
            pltpu.VMEM((IDX_ROWS, IDX_COLS), jnp.float32),
            pltpu.VMEM((HSIZE // NUM_SUBCORES,), jnp.float32),
            pltpu.VMEM_SHARED((HPAD,), jnp.float32),
            pltpu.SemaphoreType.DMA,
        ],
    )(_hist_body)
    return f(xcls, src, dst)


def _ln(h, g, b):
    mu = jnp.mean(h, axis=-1, keepdims=True)
    var = jnp.mean((h - mu) * (h - mu), axis=-1, keepdims=True)
    return (h - mu) * jax.lax.rsqrt(var + 1e-5) * g + b


def _main_body(part_ref, xc_ref, emb_ref,
               w1, b1, g1, be1, w2, b2, g2, be2, w3, b3,
               sw1, sb1, sw2, sb2, ng, nb,
               xf_ref, gm_ref):
    cnt = part_ref[0, 0] + part_ref[1, 0]
    cls = xc_ref[0]
    onehot = (cls == lax.broadcasted_iota(jnp.int32, (SEG, NCLS), 1))
    c = cnt + onehot.astype(jnp.float32)
    h = jnp.dot(c, emb_ref[...], preferred_element_type=jnp.float32)
    t = jnp.dot(h, w1[...], preferred_element_type=jnp.float32) + b1[...]
    t = jnp.maximum(_ln(t, g1[...], be1[...]), 0.0)
    t = jnp.dot(t, w2[...], preferred_element_type=jnp.float32) + b2[...]
    t = jnp.maximum(_ln(t, g2[...], be2[...]), 0.0)
    t = jnp.dot(t, w3[...], preferred_element_type=jnp.float32) + b3[...]
    t = jnp.maximum(jnp.dot(t, sw1[...], preferred_element_type=jnp.float32)
                    + sb1[...], 0.0)
    t = jnp.dot(t, sw2[...], preferred_element_type=jnp.float32) + sb2[...]
    xf = _ln(t, ng[...], nb[...])
    xf_ref[0] = xf
    gm_ref[0, 0] = jnp.mean(xf, axis=0)


def _main(parts, xcls, p):
    g = p["gin_mlp"]
    row = lambda v: v.reshape(1, -1)
    args = [
        parts.reshape(NUM_CORES, B, SEG, NCLS),
        xcls.reshape(B, SEG, 1),
        p["embedding"],
        g["W1"], row(g["b1"]), row(g["g1"]), row(g["be1"]),
        g["W2"], row(g["b2"]), row(g["g2"]), row(g["be2"]),
        g["W3"], row(g["b3"]),
        p["seq_W1"], row(p["seq_b1"]), p["seq_W2"], row(p["seq_b2"]),
        row(p["norm_g"]), row(p["norm_b"]),
    ]

    def full(a):
        nd = a.ndim
        return pl.BlockSpec(a.shape, lambda i, nd=nd: (0,) * nd)

    specs = [
        pl.BlockSpec((NUM_CORES, 1, SEG, NCLS), lambda i: (0, i, 0, 0)),
        pl.BlockSpec((1, SEG, 1), lambda i: (i, 0, 0)),
    ] + [full(a) for a in args[2:]]
    return pl.pallas_call(
        _main_body,
        grid=(B,),
        in_specs=specs,
        out_specs=[
            pl.BlockSpec((1, SEG, D), lambda i: (i, 0, 0)),
            pl.BlockSpec((1, 1, D), lambda i: (i, 0, 0)),
        ],
        out_shape=[
            jax.ShapeDtypeStruct((B, SEG, D), jnp.float32),
            jax.ShapeDtypeStruct((B, 1, D), jnp.float32),
        ],
    )(*args)


def _heads_body(gm, w1s, b1s, g1s, be1s, w2s, b2s, g2s, be2s, w3s, b3s,
                act_ref, nc_ref, ec_ref):
    x = gm[...]
    outs = []
    for i in range(4):
        t = jnp.dot(x, w1s[i], preferred_element_type=jnp.float32) + b1s[i]
        t = jnp.maximum(_ln(t, g1s[i], be1s[i]), 0.0)
        t = jnp.dot(t, w2s[i], preferred_element_type=jnp.float32) + b2s[i]
        t = jnp.maximum(_ln(t, g2s[i], be2s[i]), 0.0)
        outs.append(jnp.dot(t, w3s[i], preferred_element_type=jnp.float32)
                    + b3s[i])
    addn, exitl, ncl, ecl = outs
    zero = jnp.zeros((B, 1), jnp.float32)
    act_ref[...] = jnp.concatenate(
        [addn[:, 0:1], zero, exitl[:, 0:1]], axis=1)
    nc_ref[...] = ncl
    ec_ref[...] = ecl[:, 0:4]


def _heads(gm, p):
    mlps = [p["add_node_mlp"], p["exit_mlp"], p["node_class_mlp"],
            p["edge_class_mlp"]]
    st = lambda k: jnp.stack([m[k] for m in mlps])
    w3s = jnp.stack([
        jnp.pad(m["W3"], ((0, 0), (0, NCLS - m["W3"].shape[1])))
        for m in mlps])
    b3s = jnp.stack([jnp.pad(m["b3"], (0, NCLS - m["b3"].shape[0]))
                     for m in mlps])
    args = [gm, st("W1"), st("b1"), st("g1"), st("be1"),
            st("W2"), st("b2"), st("g2"), st("be2"), w3s, b3s]
    return pl.pallas_call(
        _heads_body,
        out_shape=[
            jax.ShapeDtypeStruct((B, 3), jnp.float32),
            jax.ShapeDtypeStruct((B, NCLS), jnp.float32),
            jax.ShapeDtypeStruct((B, 4), jnp.float32),
        ],
    )(*args)


def _scores_body(xf_ref, out_ref):
    xg = xf_ref[0]
    s = lax.dot_general(xg, xg, (((1,), (1,)), ((), ())),
                        preferred_element_type=jnp.float32)
    out_ref[0, :, pl.ds(0, SEG)] = s * np.float32(1.0 / np.sqrt(D))


def _scores(xf):
    return pl.pallas_call(
        _scores_body,
        grid=(B,),
        in_specs=[pl.BlockSpec((1, SEG, D), lambda i: (i, 0, 0))],
        out_specs=pl.BlockSpec((1, SEG, COLS), lambda i: (i, 0, 0)),
        out_shape=jax.ShapeDtypeStruct((B, SEG, COLS), jnp.float32),
    )(xf)


WLEN = COLS + 128


def _extract_body(s_ref, out_ref):
    r0 = pl.program_id(0) * ROWT

    def body(t, _):
        i = r0 + t
        rows = s_ref[:, 0, t, :]
        prow = jnp.concatenate(
            [rows, jnp.zeros((B, WLEN - COLS), jnp.float32)], axis=1)
        off = i * (SEG - 1) - (i * (i - 1)) // 2
        a = pl.multiple_of((off // 128) * 128, 128)
        delta = off - a
        q = i + 1 - delta
        shifted = pltpu.roll(prow, lax.rem(2 * WLEN - q, WLEN), axis=1)
        cur = out_ref[:, pl.ds(a, WLEN)]
        u = lax.broadcasted_iota(jnp.int32, (B, WLEN), 1)
        keep = (u >= delta) & (u < delta + (SEG - 1 - i))
        out_ref[:, pl.ds(a, WLEN)] = jnp.where(keep, shifted, cur)
        return 0

    lax.fori_loop(0, ROWT, body, 0)


def _extract(spad):
    spad = spad.reshape(B, SEG // ROWT, ROWT, COLS)
    return pl.pallas_call(
        _extract_body,
        grid=(SEG // ROWT,),
        in_specs=[pl.BlockSpec((B, 1, ROWT, COLS), lambda i: (0, i, 0, 0))],
        out_specs=pl.BlockSpec((B, OUTP), lambda i: (0, 0)),
        out_shape=jax.ShapeDtypeStruct((B, OUTP), jnp.float32),
        compiler_params=pltpu.CompilerParams(
            dimension_semantics=("arbitrary",)),
    )(spad)


def kernel(x, edge_index, ptr, params):
    xcls = x.reshape(N)
    parts = _histogram(xcls, edge_index[0], edge_index[1])
    xf, gm = _main(parts, xcls, params)
    act, ncl, ecl = _heads(gm.reshape(B, D), params)
    spad = _scores(xf)
    logits = _extract(spad)[:, :TRI]
    return act, ncl, ecl, logits


# speedup vs baseline: 8.1600x; 1.2806x over previous
import functools
import numpy as np
import jax
import jax.numpy as jnp
from jax import lax
from jax.experimental import pallas as pl
from jax.experimental.pallas import tpu as pltpu
from jax.experimental.pallas import tpu_sc as plsc

N = 10000
E = 320000
B = 8
D = 128
NCLS = 16
SEG = N // B
HSIZE = N * NCLS
HPAD = HSIZE + 8

NUM_CORES = 2
NUM_SUBCORES = 16
W = NUM_CORES * NUM_SUBCORES
EPW = E // W
IDX_COLS = 128
FULL_ROWS = EPW // IDX_COLS
IDX_ROWS = FULL_ROWS + 1

COLS = 1408
TRI = SEG * (SEG - 1) // 2
OUTP = 781952
CHAINS = 10
STRIDE = SEG // CHAINS
ROWT = 5


def _hist_body(xcls_hbm, src_hbm, dst_hbm, out_hbm,
               xcls_v, src_v, dst_v, idx_v, upd_v, zero_v, hist_sh, sem):
    c = lax.axis_index("c")
    s = lax.axis_index("s")
    wid = c * NUM_SUBCORES + s
    base = wid * EPW
    sbase = s * (HSIZE // NUM_SUBCORES)

    pltpu.sync_copy(xcls_hbm, xcls_v)
    pltpu.sync_copy(src_hbm.at[pl.ds(base, EPW)], src_v)
    pltpu.sync_copy(dst_hbm.at[pl.ds(base, EPW)], dst_v)

    ones16 = jnp.full((16,), 1.0, jnp.float32)
    zeros16 = jnp.zeros((16,), jnp.float32)
    trash16 = jnp.full((16,), HSIZE, jnp.int32)

    def zfill(i, _):
        zero_v[pl.ds(i * 16, 16)] = zeros16
        return 0
    lax.fori_loop(0, (HSIZE // NUM_SUBCORES) // 16, zfill, 0)
    pltpu.sync_copy(zero_v, hist_sh.at[pl.ds(sbase, HSIZE // NUM_SUBCORES)])

    def ifill(i, _):
        r = i // 8
        col = (i % 8) * 16
        upd_v[r, pl.ds(col, 16)] = ones16
        idx_v[r, pl.ds(col, 16)] = trash16
        return 0
    lax.fori_loop(0, IDX_ROWS * 8, ifill, 0)

    def efill(j, _):
        sv = src_v[pl.ds(j * 16, 16)]
        dv = dst_v[pl.ds(j * 16, 16)]
        cls = plsc.load_gather(xcls_v, [sv])
        flat = dv * 16 + cls
        r = j // 8
        col = (j % 8) * 16
        idx_v[r, pl.ds(col, 16)] = flat
        return 0
    lax.fori_loop(0, EPW // 16, efill, 0)

    plsc.subcore_barrier()

    def scat(r, _):
        pltpu.sync_copy(upd_v.at[r], hist_sh.at[idx_v.at[r]], add=True)
        return 0
    lax.fori_loop(0, IDX_ROWS, scat, 0)

    plsc.subcore_barrier()

    pltpu.sync_copy(hist_sh.at[pl.ds(sbase, HSIZE // NUM_SUBCORES)], zero_v)
    pltpu.sync_copy(zero_v,
                    out_hbm.at[pl.ds(c * HSIZE + sbase, HSIZE // NUM_SUBCORES)])


def _histogram(xcls, src, dst):
    mesh = plsc.VectorSubcoreMesh(core_axis_name="c", subcore_axis_name="s")
    f = functools.partial(
        pl.kernel,
        mesh=mesh,
        compiler_params=pltpu.CompilerParams(needs_layout_passes=False),
        out_type=jax.ShapeDtypeStruct((NUM_CORES * HSIZE,), jnp.float32),
        scratch_types=[
            pltpu.VMEM((N,), jnp.int32),
            pltpu.VMEM((EPW,), jnp.int32),
            pltpu.VMEM((EPW,), jnp.int32),
            pltpu.VMEM((IDX_ROWS, IDX_COLS), jnp.int32),
            pltpu.VMEM((IDX_ROWS, IDX_COLS), jnp.float32),
            pltpu.VMEM((HSIZE // NUM_SUBCORES,), jnp.float32),
            pltpu.VMEM_SHARED((HPAD,), jnp.float32),
            pltpu.SemaphoreType.DMA,
        ],
    )(_hist_body)
    return f(xcls, src, dst)


def _ln(h, g, b):
    mu = jnp.mean(h, axis=-1, keepdims=True)
    var = jnp.mean((h - mu) * (h - mu), axis=-1, keepdims=True)
    return (h - mu) * jax.lax.rsqrt(var + 1e-5) * g + b


def _main_body(part_ref, xc_ref, emb_ref,
               w1, b1, g1, be1, w2, b2, g2, be2, w3, b3,
               sw1, sb1, sw2, sb2, ng, nb,
               xf_ref, gm_ref):
    cnt = part_ref[0, 0] + part_ref[1, 0]
    cls = xc_ref[0]
    onehot = (cls == lax.broadcasted_iota(jnp.int32, (SEG, NCLS), 1))
    c = cnt + onehot.astype(jnp.float32)
    h = jnp.dot(c, emb_ref[...], preferred_element_type=jnp.float32)
    t = jnp.dot(h, w1[...], preferred_element_type=jnp.float32) + b1[...]
    t = jnp.maximum(_ln(t, g1[...], be1[...]), 0.0)
    t = jnp.dot(t, w2[...], preferred_element_type=jnp.float32) + b2[...]
    t = jnp.maximum(_ln(t, g2[...], be2[...]), 0.0)
    t = jnp.dot(t, w3[...], preferred_element_type=jnp.float32) + b3[...]
    t = jnp.maximum(jnp.dot(t, sw1[...], preferred_element_type=jnp.float32)
                    + sb1[...], 0.0)
    t = jnp.dot(t, sw2[...], preferred_element_type=jnp.float32) + sb2[...]
    xf = _ln(t, ng[...], nb[...])
    xf_ref[0] = xf
    gm_ref[0, 0] = jnp.mean(xf, axis=0)


def _main(parts, xcls, p):
    g = p["gin_mlp"]
    row = lambda v: v.reshape(1, -1)
    args = [
        parts.reshape(NUM_CORES, B, SEG, NCLS),
        xcls.reshape(B, SEG, 1),
        p["embedding"],
        g["W1"], row(g["b1"]), row(g["g1"]), row(g["be1"]),
        g["W2"], row(g["b2"]), row(g["g2"]), row(g["be2"]),
        g["W3"], row(g["b3"]),
        p["seq_W1"], row(p["seq_b1"]), p["seq_W2"], row(p["seq_b2"]),
        row(p["norm_g"]), row(p["norm_b"]),
    ]

    def full(a):
        nd = a.ndim
        return pl.BlockSpec(a.shape, lambda i, nd=nd: (0,) * nd)

    specs = [
        pl.BlockSpec((NUM_CORES, 1, SEG, NCLS), lambda i: (0, i, 0, 0)),
        pl.BlockSpec((1, SEG, 1), lambda i: (i, 0, 0)),
    ] + [full(a) for a in args[2:]]
    return pl.pallas_call(
        _main_body,
        grid=(B,),
        in_specs=specs,
        out_specs=[
            pl.BlockSpec((1, SEG, D), lambda i: (i, 0, 0)),
            pl.BlockSpec((1, 1, D), lambda i: (i, 0, 0)),
        ],
        out_shape=[
            jax.ShapeDtypeStruct((B, SEG, D), jnp.float32),
            jax.ShapeDtypeStruct((B, 1, D), jnp.float32),
        ],
    )(*args)


def _heads_body(gm, w1s, b1s, g1s, be1s, w2s, b2s, g2s, be2s, w3s, b3s,
                act_ref, nc_ref, ec_ref):
    x = gm[...]
    outs = []
    for i in range(4):
        t = jnp.dot(x, w1s[i], preferred_element_type=jnp.float32) + b1s[i]
        t = jnp.maximum(_ln(t, g1s[i], be1s[i]), 0.0)
        t = jnp.dot(t, w2s[i], preferred_element_type=jnp.float32) + b2s[i]
        t = jnp.maximum(_ln(t, g2s[i], be2s[i]), 0.0)
        outs.append(jnp.dot(t, w3s[i], preferred_element_type=jnp.float32)
                    + b3s[i])
    addn, exitl, ncl, ecl = outs
    zero = jnp.zeros((B, 1), jnp.float32)
    act_ref[...] = jnp.concatenate(
        [addn[:, 0:1], zero, exitl[:, 0:1]], axis=1)
    nc_ref[...] = ncl
    ec_ref[...] = ecl[:, 0:4]


def _heads(gm, p):
    mlps = [p["add_node_mlp"], p["exit_mlp"], p["node_class_mlp"],
            p["edge_class_mlp"]]
    st = lambda k: jnp.stack([m[k] for m in mlps])
    w3s = jnp.stack([
        jnp.pad(m["W3"], ((0, 0), (0, NCLS - m["W3"].shape[1])))
        for m in mlps])
    b3s = jnp.stack([jnp.pad(m["b3"], (0, NCLS - m["b3"].shape[0]))
                     for m in mlps])
    args = [gm, st("W1"), st("b1"), st("g1"), st("be1"),
            st("W2"), st("b2"), st("g2"), st("be2"), w3s, b3s]
    return pl.pallas_call(
        _heads_body,
        out_shape=[
            jax.ShapeDtypeStruct((B, 3), jnp.float32),
            jax.ShapeDtypeStruct((B, NCLS), jnp.float32),
            jax.ShapeDtypeStruct((B, 4), jnp.float32),
        ],
    )(*args)


def _scores_body(xf_ref, out_ref):
    xg = xf_ref[0]
    s = lax.dot_general(xg, xg, (((1,), (1,)), ((), ())),
                        preferred_element_type=jnp.float32)
    out_ref[0, :, pl.ds(0, SEG)] = s * np.float32(1.0 / np.sqrt(D))


def _scores(xf):
    return pl.pallas_call(
        _scores_body,
        grid=(B,),
        in_specs=[pl.BlockSpec((1, SEG, D), lambda i: (i, 0, 0))],
        out_specs=pl.BlockSpec((1, SEG, COLS), lambda i: (i, 0, 0)),
        out_shape=jax.ShapeDtypeStruct((B, SEG, COLS), jnp.float32),
    )(xf)


WLEN = COLS


def _extract_body(s_ref, out_ref):
    r0 = pl.program_id(0) * ROWT
    u = lax.broadcasted_iota(jnp.int32, (B, WLEN), 1)

    def body(t, _):
        x = r0 + t
        for c in range(CHAINS):
            i = c * STRIDE + x
            prow = s_ref[:, c, 0, t, :]
            off = i * (SEG - 1) - (i * (i - 1)) // 2
            a = pl.multiple_of((off // 128) * 128, 128)
            delta = off - a
            q = i + 1 - delta
            shifted = pltpu.roll(prow, lax.rem(2 * WLEN - q, WLEN), axis=1)
            cur = out_ref[:, pl.ds(a, WLEN)]
            keep = (u >= delta) & (u < delta + (SEG - 1 - i))
            out_ref[:, pl.ds(a, WLEN)] = jnp.where(keep, shifted, cur)
        return 0

    lax.fori_loop(0, ROWT, body, 0)


def _extract(spad):
    spad = spad.reshape(B, CHAINS, STRIDE // ROWT, ROWT, COLS)
    return pl.pallas_call(
        _extract_body,
        grid=(STRIDE // ROWT,),
        in_specs=[pl.BlockSpec((B, CHAINS, 1, ROWT, COLS),
                               lambda i: (0, 0, i, 0, 0))],
        out_specs=pl.BlockSpec((B, OUTP), lambda i: (0, 0)),
        out_shape=jax.ShapeDtypeStruct((B, OUTP), jnp.float32),
        compiler_params=pltpu.CompilerParams(
            dimension_semantics=("arbitrary",)),
    )(spad)


def kernel(x, edge_index, ptr, params):
    xcls = x.reshape(N)
    parts = _histogram(xcls, edge_index[0], edge_index[1])
    xf, gm = _main(parts, xcls, params)
    act, ncl, ecl = _heads(gm.reshape(B, D), params)
    spad = _scores(xf)
    logits = _extract(spad)[:, :TRI]
    return act, ncl, ecl, logits


# speedup vs baseline: 8.3638x; 1.0250x over previous
import functools
import numpy as np
import jax
import jax.numpy as jnp
from jax import lax
from jax.experimental import pallas as pl
from jax.experimental.pallas import tpu as pltpu
from jax.experimental.pallas import tpu_sc as plsc

N = 10000
E = 320000
B = 8
D = 128
NCLS = 16
SEG = N // B
HSIZE = N * NCLS
HPAD = HSIZE + 8

NUM_CORES = 2
NUM_SUBCORES = 16
W = NUM_CORES * NUM_SUBCORES
EPW = E // W
IDX_COLS = 128
FULL_ROWS = EPW // IDX_COLS
IDX_ROWS = FULL_ROWS + 1

COLS = 1408
TRI = SEG * (SEG - 1) // 2
OUTP = 781952
CHAINS = 10
STRIDE = SEG // CHAINS
ROWT = 5


def _hist_body(xcls_hbm, src_hbm, dst_hbm, out_hbm,
               xcls_v, src_v, dst_v, idx_v, upd_v, zero_v, hist_sh, sem):
    c = lax.axis_index("c")
    s = lax.axis_index("s")
    wid = c * NUM_SUBCORES + s
    base = wid * EPW
    sbase = s * (HSIZE // NUM_SUBCORES)

    pltpu.sync_copy(xcls_hbm, xcls_v)
    pltpu.sync_copy(src_hbm.at[pl.ds(base, EPW)], src_v)
    pltpu.sync_copy(dst_hbm.at[pl.ds(base, EPW)], dst_v)

    ones16 = jnp.full((16,), 1.0, jnp.float32)
    zeros16 = jnp.zeros((16,), jnp.float32)
    trash16 = jnp.full((16,), HSIZE, jnp.int32)

    def zfill(i, _):
        zero_v[pl.ds(i * 16, 16)] = zeros16
        return 0
    lax.fori_loop(0, (HSIZE // NUM_SUBCORES) // 16, zfill, 0)
    pltpu.sync_copy(zero_v, hist_sh.at[pl.ds(sbase, HSIZE // NUM_SUBCORES)])

    def ifill(i, _):
        r = i // 8
        col = (i % 8) * 16
        upd_v[r, pl.ds(col, 16)] = ones16
        idx_v[r, pl.ds(col, 16)] = trash16
        return 0
    lax.fori_loop(0, IDX_ROWS * 8, ifill, 0)

    def efill(j, _):
        sv = src_v[pl.ds(j * 16, 16)]
        dv = dst_v[pl.ds(j * 16, 16)]
        cls = plsc.load_gather(xcls_v, [sv])
        flat = dv * 16 + cls
        r = j // 8
        col = (j % 8) * 16
        idx_v[r, pl.ds(col, 16)] = flat
        return 0
    lax.fori_loop(0, EPW // 16, efill, 0)

    plsc.subcore_barrier()

    def scat(r, _):
        pltpu.async_copy(upd_v.at[r], hist_sh.at[idx_v.at[r]], sem, add=True)
        return 0
    lax.fori_loop(0, IDX_ROWS, scat, 0)

    def drain(r, _):
        pltpu.make_async_copy(upd_v.at[0], hist_sh.at[idx_v.at[0]], sem).wait()
        return 0
    lax.fori_loop(0, IDX_ROWS, drain, 0)

    plsc.subcore_barrier()

    pltpu.sync_copy(hist_sh.at[pl.ds(sbase, HSIZE // NUM_SUBCORES)], zero_v)
    pltpu.sync_copy(zero_v,
                    out_hbm.at[pl.ds(c * HSIZE + sbase, HSIZE // NUM_SUBCORES)])


def _histogram(xcls, src, dst):
    mesh = plsc.VectorSubcoreMesh(core_axis_name="c", subcore_axis_name="s")
    f = functools.partial(
        pl.kernel,
        mesh=mesh,
        compiler_params=pltpu.CompilerParams(needs_layout_passes=False),
        out_type=jax.ShapeDtypeStruct((NUM_CORES * HSIZE,), jnp.float32),
        scratch_types=[
            pltpu.VMEM((N,), jnp.int32),
            pltpu.VMEM((EPW,), jnp.int32),
            pltpu.VMEM((EPW,), jnp.int32),
            pltpu.VMEM((IDX_ROWS, IDX_COLS), jnp.int32),
            pltpu.VMEM((IDX_ROWS, IDX_COLS), jnp.float32),
            pltpu.VMEM((HSIZE // NUM_SUBCORES,), jnp.float32),
            pltpu.VMEM_SHARED((HPAD,), jnp.float32),
            pltpu.SemaphoreType.DMA,
        ],
    )(_hist_body)
    return f(xcls, src, dst)


def _ln(h, g, b):
    mu = jnp.mean(h, axis=-1, keepdims=True)
    var = jnp.mean((h - mu) * (h - mu), axis=-1, keepdims=True)
    return (h - mu) * jax.lax.rsqrt(var + 1e-5) * g + b


def _main_body(part_ref, xc_ref, emb_ref,
               w1, b1, g1, be1, w2, b2, g2, be2, w3, b3,
               sw1, sb1, sw2, sb2, ng, nb,
               xf_ref, gm_ref):
    cnt = part_ref[0, 0] + part_ref[1, 0]
    cls = xc_ref[0]
    onehot = (cls == lax.broadcasted_iota(jnp.int32, (SEG, NCLS), 1))
    c = cnt + onehot.astype(jnp.float32)
    h = jnp.dot(c, emb_ref[...], preferred_element_type=jnp.float32)
    t = jnp.dot(h, w1[...], preferred_element_type=jnp.float32) + b1[...]
    t = jnp.maximum(_ln(t, g1[...], be1[...]), 0.0)
    t = jnp.dot(t, w2[...], preferred_element_type=jnp.float32) + b2[...]
    t = jnp.maximum(_ln(t, g2[...], be2[...]), 0.0)
    t = jnp.dot(t, w3[...], preferred_element_type=jnp.float32) + b3[...]
    t = jnp.maximum(jnp.dot(t, sw1[...], preferred_element_type=jnp.float32)
                    + sb1[...], 0.0)
    t = jnp.dot(t, sw2[...], preferred_element_type=jnp.float32) + sb2[...]
    xf = _ln(t, ng[...], nb[...])
    xf_ref[0] = xf
    gm_ref[0, 0] = jnp.mean(xf, axis=0)


def _main(parts, xcls, p):
    g = p["gin_mlp"]
    row = lambda v: v.reshape(1, -1)
    args = [
        parts.reshape(NUM_CORES, B, SEG, NCLS),
        xcls.reshape(B, SEG, 1),
        p["embedding"],
        g["W1"], row(g["b1"]), row(g["g1"]), row(g["be1"]),
        g["W2"], row(g["b2"]), row(g["g2"]), row(g["be2"]),
        g["W3"], row(g["b3"]),
        p["seq_W1"], row(p["seq_b1"]), p["seq_W2"], row(p["seq_b2"]),
        row(p["norm_g"]), row(p["norm_b"]),
    ]

    def full(a):
        nd = a.ndim
        return pl.BlockSpec(a.shape, lambda i, nd=nd: (0,) * nd)

    specs = [
        pl.BlockSpec((NUM_CORES, 1, SEG, NCLS), lambda i: (0, i, 0, 0)),
        pl.BlockSpec((1, SEG, 1), lambda i: (i, 0, 0)),
    ] + [full(a) for a in args[2:]]
    return pl.pallas_call(
        _main_body,
        grid=(B,),
        in_specs=specs,
        out_specs=[
            pl.BlockSpec((1, SEG, D), lambda i: (i, 0, 0)),
            pl.BlockSpec((1, 1, D), lambda i: (i, 0, 0)),
        ],
        out_shape=[
            jax.ShapeDtypeStruct((B, SEG, D), jnp.float32),
            jax.ShapeDtypeStruct((B, 1, D), jnp.float32),
        ],
    )(*args)


def _scores_body(xf_ref, gm, w1s, b1s, g1s, be1s, w2s, b2s, g2s, be2s,
                 w3s, b3s, out_ref, act_ref, nc_ref, ec_ref):
    @pl.when(pl.program_id(0) == 0)
    def _():
        x = gm[...]
        outs = []
        for i in range(4):
            t = jnp.dot(x, w1s[i], preferred_element_type=jnp.float32) + b1s[i]
            t = jnp.maximum(_ln(t, g1s[i], be1s[i]), 0.0)
            t = jnp.dot(t, w2s[i], preferred_element_type=jnp.float32) + b2s[i]
            t = jnp.maximum(_ln(t, g2s[i], be2s[i]), 0.0)
            outs.append(jnp.dot(t, w3s[i], preferred_element_type=jnp.float32)
                        + b3s[i])
        addn, exitl, ncl, ecl = outs
        zero = jnp.zeros((B, 1), jnp.float32)
        act_ref[...] = jnp.concatenate(
            [addn[:, 0:1], zero, exitl[:, 0:1]], axis=1)
        nc_ref[...] = ncl
        ec_ref[...] = ecl[:, 0:4]

    xg = xf_ref[0]
    s = lax.dot_general(xg, xg, (((1,), (1,)), ((), ())),
                        preferred_element_type=jnp.float32)
    out_ref[0, :, pl.ds(0, SEG)] = s * np.float32(1.0 / np.sqrt(D))


def _scores(xf, gm, p):
    mlps = [p["add_node_mlp"], p["exit_mlp"], p["node_class_mlp"],
            p["edge_class_mlp"]]
    st = lambda k: jnp.stack([m[k] for m in mlps])
    w3s = jnp.stack([
        jnp.pad(m["W3"], ((0, 0), (0, NCLS - m["W3"].shape[1])))
        for m in mlps])
    b3s = jnp.stack([jnp.pad(m["b3"], (0, NCLS - m["b3"].shape[0]))
                     for m in mlps])
    args = [xf, gm, st("W1"), st("b1"), st("g1"), st("be1"),
            st("W2"), st("b2"), st("g2"), st("be2"), w3s, b3s]

    def full(a):
        nd = a.ndim
        return pl.BlockSpec(a.shape, lambda i, nd=nd: (0,) * nd)

    specs = [pl.BlockSpec((1, SEG, D), lambda i: (i, 0, 0))]
    specs += [full(a) for a in args[1:]]
    return pl.pallas_call(
        _scores_body,
        grid=(B,),
        in_specs=specs,
        out_specs=[
            pl.BlockSpec((1, SEG, COLS), lambda i: (i, 0, 0)),
            pl.BlockSpec((B, 3), lambda i: (0, 0)),
            pl.BlockSpec((B, NCLS), lambda i: (0, 0)),
            pl.BlockSpec((B, 4), lambda i: (0, 0)),
        ],
        out_shape=[
            jax.ShapeDtypeStruct((B, SEG, COLS), jnp.float32),
            jax.ShapeDtypeStruct((B, 3), jnp.float32),
            jax.ShapeDtypeStruct((B, NCLS), jnp.float32),
            jax.ShapeDtypeStruct((B, 4), jnp.float32),
        ],
    )(*args)


WLEN = COLS


def _extract_body(s_ref, out_ref):
    r0 = pl.program_id(0) * ROWT
    u = lax.broadcasted_iota(jnp.int32, (B, WLEN), 1)

    def body(t, _):
        x = r0 + t
        for c in range(CHAINS):
            i = c * STRIDE + x
            prow = s_ref[:, c, 0, t, :]
            off = i * (SEG - 1) - (i * (i - 1)) // 2
            a = pl.multiple_of((off // 128) * 128, 128)
            delta = off - a
            q = i + 1 - delta
            shifted = pltpu.roll(prow, lax.rem(2 * WLEN - q, WLEN), axis=1)
            cur = out_ref[:, pl.ds(a, WLEN)]
            keep = (u >= delta) & (u < delta + (SEG - 1 - i))
            out_ref[:, pl.ds(a, WLEN)] = jnp.where(keep, shifted, cur)
        return 0

    lax.fori_loop(0, ROWT, body, 0)


def _extract(spad):
    spad = spad.reshape(B, CHAINS, STRIDE // ROWT, ROWT, COLS)
    return pl.pallas_call(
        _extract_body,
        grid=(STRIDE // ROWT,),
        in_specs=[pl.BlockSpec((B, CHAINS, 1, ROWT, COLS),
                               lambda i: (0, 0, i, 0, 0))],
        out_specs=pl.BlockSpec((B, OUTP), lambda i: (0, 0)),
        out_shape=jax.ShapeDtypeStruct((B, OUTP), jnp.float32),
        compiler_params=pltpu.CompilerParams(
            dimension_semantics=("arbitrary",)),
    )(spad)


def kernel(x, edge_index, ptr, params):
    xcls = x.reshape(N)
    parts = _histogram(xcls, edge_index[0], edge_index[1])
    xf, gm = _main(parts, xcls, params)
    spad, act, ncl, ecl = _scores(xf, gm.reshape(B, D), params)
    logits = _extract(spad)[:, :TRI]
    return act, ncl, ecl, logits


# speedup vs baseline: 8.4445x; 1.0097x over previous
import functools
import numpy as np
import jax
import jax.numpy as jnp
from jax import lax
from jax.experimental import pallas as pl
from jax.experimental.pallas import tpu as pltpu
from jax.experimental.pallas import tpu_sc as plsc

N = 10000
E = 320000
B = 8
D = 128
NCLS = 16
SEG = N // B
HSIZE = N * NCLS
HPAD = HSIZE + 8

NUM_CORES = 2
NUM_SUBCORES = 16
W = NUM_CORES * NUM_SUBCORES
EPW = E // W
IDX_COLS = 128
FULL_ROWS = EPW // IDX_COLS
IDX_ROWS = FULL_ROWS + 1

COLS = 1408
TRI = SEG * (SEG - 1) // 2
OUTP = 781952
CHAINS = 10
STRIDE = SEG // CHAINS
ROWT = 5


def _hist_body(xcls_hbm, src_hbm, dst_hbm, out_hbm,
               xcls_v, src_v, dst_v, idx_v, upd_v, zero_v, hist_sh, sem):
    c = lax.axis_index("c")
    s = lax.axis_index("s")
    wid = c * NUM_SUBCORES + s
    base = wid * EPW
    sbase = s * (HSIZE // NUM_SUBCORES)

    pltpu.sync_copy(xcls_hbm, xcls_v)
    pltpu.sync_copy(src_hbm.at[pl.ds(base, EPW)], src_v)
    pltpu.sync_copy(dst_hbm.at[pl.ds(base, EPW)], dst_v)

    ones16 = jnp.full((16,), 1.0, jnp.float32)
    zeros16 = jnp.zeros((16,), jnp.float32)
    trash16 = jnp.full((16,), HSIZE, jnp.int32)

    def zfill(i, _):
        zero_v[pl.ds(i * 16, 16)] = zeros16
        return 0
    lax.fori_loop(0, (HSIZE // NUM_SUBCORES) // 16, zfill, 0)
    pltpu.sync_copy(zero_v, hist_sh.at[pl.ds(sbase, HSIZE // NUM_SUBCORES)])

    def ifill(i, _):
        r = i // 8
        col = (i % 8) * 16
        upd_v[r, pl.ds(col, 16)] = ones16
        idx_v[r, pl.ds(col, 16)] = trash16
        return 0
    lax.fori_loop(0, IDX_ROWS * 8, ifill, 0)

    def efill(j, _):
        sv = src_v[pl.ds(j * 16, 16)]
        dv = dst_v[pl.ds(j * 16, 16)]
        cls = plsc.load_gather(xcls_v, [sv])
        flat = dv * 16 + cls
        r = j // 8
        col = (j % 8) * 16
        idx_v[r, pl.ds(col, 16)] = flat
        return 0
    lax.fori_loop(0, EPW // 16, efill, 0)

    plsc.subcore_barrier()

    def scat(r, _):
        pltpu.async_copy(upd_v.at[r], hist_sh.at[idx_v.at[r]], sem, add=True)
        return 0
    lax.fori_loop(0, IDX_ROWS, scat, 0)

    def drain(r, _):
        pltpu.make_async_copy(upd_v.at[0], hist_sh.at[idx_v.at[0]], sem).wait()
        return 0
    lax.fori_loop(0, IDX_ROWS, drain, 0)

    plsc.subcore_barrier()

    pltpu.sync_copy(hist_sh.at[pl.ds(sbase, HSIZE // NUM_SUBCORES)], zero_v)
    pltpu.sync_copy(zero_v,
                    out_hbm.at[pl.ds(c * HSIZE + sbase, HSIZE // NUM_SUBCORES)])


def _histogram(xcls, src, dst):
    mesh = plsc.VectorSubcoreMesh(core_axis_name="c", subcore_axis_name="s")
    f = functools.partial(
        pl.kernel,
        mesh=mesh,
        compiler_params=pltpu.CompilerParams(needs_layout_passes=False),
        out_type=jax.ShapeDtypeStruct((NUM_CORES * HSIZE,), jnp.float32),
        scratch_types=[
            pltpu.VMEM((N,), jnp.int32),
            pltpu.VMEM((EPW,), jnp.int32),
            pltpu.VMEM((EPW,), jnp.int32),
            pltpu.VMEM((IDX_ROWS, IDX_COLS), jnp.int32),
            pltpu.VMEM((IDX_ROWS, IDX_COLS), jnp.float32),
            pltpu.VMEM((HSIZE // NUM_SUBCORES,), jnp.float32),
            pltpu.VMEM_SHARED((HPAD,), jnp.float32),
            pltpu.SemaphoreType.DMA,
        ],
    )(_hist_body)
    return f(xcls, src, dst)


def _ln(h, g, b):
    mu = jnp.mean(h, axis=-1, keepdims=True)
    var = jnp.mean((h - mu) * (h - mu), axis=-1, keepdims=True)
    return (h - mu) * jax.lax.rsqrt(var + 1e-5) * g + b


def _main_body(part_ref, xc_ref, emb_ref,
               w1, b1, g1, be1, w2, b2, g2, be2, w3, b3,
               sw1, sb1, sw2, sb2, ng, nb,
               w1s, b1s, g1s, be1s, w2s, b2s, g2s, be2s, w3s, b3s,
               s_ref, act_ref, nc_ref, ec_ref):
    cnt = part_ref[0, 0] + part_ref[1, 0]
    cls = xc_ref[0]
    onehot = (cls == lax.broadcasted_iota(jnp.int32, (SEG, NCLS), 1))
    c = cnt + onehot.astype(jnp.float32)
    h = jnp.dot(c, emb_ref[...], preferred_element_type=jnp.float32)
    t = jnp.dot(h, w1[...], preferred_element_type=jnp.float32) + b1[...]
    t = jnp.maximum(_ln(t, g1[...], be1[...]), 0.0)
    t = jnp.dot(t, w2[...], preferred_element_type=jnp.float32) + b2[...]
    t = jnp.maximum(_ln(t, g2[...], be2[...]), 0.0)
    t = jnp.dot(t, w3[...], preferred_element_type=jnp.float32) + b3[...]
    t = jnp.maximum(jnp.dot(t, sw1[...], preferred_element_type=jnp.float32)
                    + sb1[...], 0.0)
    t = jnp.dot(t, sw2[...], preferred_element_type=jnp.float32) + sb2[...]
    xf = _ln(t, ng[...], nb[...])

    s = lax.dot_general(xf, xf, (((1,), (1,)), ((), ())),
                        preferred_element_type=jnp.float32)
    s_ref[0, :, pl.ds(0, SEG)] = s * np.float32(1.0 / np.sqrt(D))

    gm = jnp.mean(xf, axis=0, keepdims=True)
    outs = []
    for i in range(4):
        hh = jnp.dot(gm, w1s[i], preferred_element_type=jnp.float32) + b1s[i]
        hh = jnp.maximum(_ln(hh, g1s[i], be1s[i]), 0.0)
        hh = jnp.dot(hh, w2s[i], preferred_element_type=jnp.float32) + b2s[i]
        hh = jnp.maximum(_ln(hh, g2s[i], be2s[i]), 0.0)
        outs.append(jnp.dot(hh, w3s[i], preferred_element_type=jnp.float32)
                    + b3s[i])
    addn, exitl, ncl, ecl = outs
    zero = jnp.zeros((1, 1), jnp.float32)
    act_ref[0] = jnp.concatenate([addn[:, 0:1], zero, exitl[:, 0:1]], axis=1)
    nc_ref[0] = ncl
    ec_ref[0] = ecl[:, 0:4]


def _main(parts, xcls, p):
    g = p["gin_mlp"]
    row = lambda v: v.reshape(1, -1)
    mlps = [p["add_node_mlp"], p["exit_mlp"], p["node_class_mlp"],
            p["edge_class_mlp"]]
    st = lambda k: jnp.stack([m[k] for m in mlps])
    w3s = jnp.stack([
        jnp.pad(m["W3"], ((0, 0), (0, NCLS - m["W3"].shape[1])))
        for m in mlps])
    b3s = jnp.stack([jnp.pad(m["b3"], (0, NCLS - m["b3"].shape[0]))
                     for m in mlps])
    args = [
        parts.reshape(NUM_CORES, B, SEG, NCLS),
        xcls.reshape(B, SEG, 1),
        p["embedding"],
        g["W1"], row(g["b1"]), row(g["g1"]), row(g["be1"]),
        g["W2"], row(g["b2"]), row(g["g2"]), row(g["be2"]),
        g["W3"], row(g["b3"]),
        p["seq_W1"], row(p["seq_b1"]), p["seq_W2"], row(p["seq_b2"]),
        row(p["norm_g"]), row(p["norm_b"]),
        st("W1"), st("b1"), st("g1"), st("be1"),
        st("W2"), st("b2"), st("g2"), st("be2"), w3s, b3s,
    ]

    def full(a):
        nd = a.ndim
        return pl.BlockSpec(a.shape, lambda i, nd=nd: (0,) * nd)

    specs = [
        pl.BlockSpec((NUM_CORES, 1, SEG, NCLS), lambda i: (0, i, 0, 0)),
        pl.BlockSpec((1, SEG, 1), lambda i: (i, 0, 0)),
    ] + [full(a) for a in args[2:]]
    spad, act, nc, ec = pl.pallas_call(
        _main_body,
        grid=(B,),
        in_specs=specs,
        out_specs=[
            pl.BlockSpec((1, SEG, COLS), lambda i: (i, 0, 0)),
            pl.BlockSpec((1, 1, 3), lambda i: (i, 0, 0)),
            pl.BlockSpec((1, 1, NCLS), lambda i: (i, 0, 0)),
            pl.BlockSpec((1, 1, 4), lambda i: (i, 0, 0)),
        ],
        out_shape=[
            jax.ShapeDtypeStruct((B, SEG, COLS), jnp.float32),
            jax.ShapeDtypeStruct((B, 1, 3), jnp.float32),
            jax.ShapeDtypeStruct((B, 1, NCLS), jnp.float32),
            jax.ShapeDtypeStruct((B, 1, 4), jnp.float32),
        ],
    )(*args)
    return spad, act.reshape(B, 3), nc.reshape(B, NCLS), ec.reshape(B, 4)


WLEN = COLS


def _extract_body(s_ref, out_ref):
    r0 = pl.program_id(0) * ROWT
    u = lax.broadcasted_iota(jnp.int32, (B, WLEN), 1)

    def body(t, _):
        x = r0 + t
        for c in range(CHAINS):
            i = c * STRIDE + x
            prow = s_ref[:, c, 0, t, :]
            off = i * (SEG - 1) - (i * (i - 1)) // 2
            a = pl.multiple_of((off // 128) * 128, 128)
            delta = off - a
            q = i + 1 - delta
            shifted = pltpu.roll(prow, lax.rem(2 * WLEN - q, WLEN), axis=1)
            cur = out_ref[:, pl.ds(a, WLEN)]
            keep = (u >= delta) & (u < delta + (SEG - 1 - i))
            out_ref[:, pl.ds(a, WLEN)] = jnp.where(keep, shifted, cur)
        return 0

    lax.fori_loop(0, ROWT, body, 0)


def _extract(spad):
    spad = spad.reshape(B, CHAINS, STRIDE // ROWT, ROWT, COLS)
    return pl.pallas_call(
        _extract_body,
        grid=(STRIDE // ROWT,),
        in_specs=[pl.BlockSpec((B, CHAINS, 1, ROWT, COLS),
                               lambda i: (0, 0, i, 0, 0))],
        out_specs=pl.BlockSpec((B, OUTP), lambda i: (0, 0)),
        out_shape=jax.ShapeDtypeStruct((B, OUTP), jnp.float32),
        compiler_params=pltpu.CompilerParams(
            dimension_semantics=("arbitrary",)),
    )(spad)


def kernel(x, edge_index, ptr, params):
    xcls = x.reshape(N)
    parts = _histogram(xcls, edge_index[0], edge_index[1])
    spad, act, ncl, ecl = _main(parts, xcls, params)
    logits = _extract(spad)[:, :TRI]
    return act, ncl, ecl, logits
